# baseline (speedup 1.0000x reference)
import functools

import jax
import jax.numpy as jnp
from jax import lax
from jax.experimental import pallas as pl
from jax.experimental.pallas import tpu as pltpu
from jax.experimental.pallas import tpu_sc as plsc

BATCH = 16384
EMBED = 32
LANES = 16
NUM_CORES = 2
NUM_SUBCORES = 16
NUM_WORKERS = NUM_CORES * NUM_SUBCORES
B_PER_W = BATCH // NUM_WORKERS
CHUNK = 128
NCHUNKS = B_PER_W // CHUNK


@functools.partial(
    pl.kernel,
    mesh=plsc.VectorSubcoreMesh(core_axis_name="c", subcore_axis_name="s"),
    out_type=jax.ShapeDtypeStruct((BATCH,), jnp.float32),
    compiler_params=pltpu.CompilerParams(
        needs_layout_passes=False, use_tc_tiling_on_sc=False),
    scratch_types=[
        pltpu.VMEM((NCHUNKS, CHUNK), jnp.int32),
        pltpu.VMEM((NCHUNKS, CHUNK), jnp.int32),
        pltpu.VMEM((B_PER_W, EMBED), jnp.float32),
        pltpu.VMEM((B_PER_W, EMBED), jnp.float32),
        pltpu.VMEM((B_PER_W,), jnp.float32),
        pltpu.SemaphoreType.DMA,
    ],
)
def _sc_dot(user_t, item_t, uid_h, iid_h, out_h,
            uidx, iidx, urows, irows, outv, sem):
    wid = lax.axis_index("s") * NUM_CORES + lax.axis_index("c")
    base = wid * B_PER_W

    pltpu.sync_copy(uid_h.at[pl.ds(wid * NCHUNKS, NCHUNKS)], uidx)
    pltpu.sync_copy(iid_h.at[pl.ds(wid * NCHUNKS, NCHUNKS)], iidx)

    copies = []
    for j in range(NCHUNKS):
        copies.append(pltpu.async_copy(
            user_t.at[uidx.at[j]], urows.at[pl.ds(j * CHUNK, CHUNK)], sem))
        copies.append(pltpu.async_copy(
            item_t.at[iidx.at[j]], irows.at[pl.ds(j * CHUNK, CHUNK)], sem))
    for c in copies:
        c.wait()

    lane_iota = lax.iota(jnp.int32, LANES)

    def body(g, carry):
        rbase = g * LANES
        acc = jnp.zeros((LANES,), jnp.float32)
        for k in range(LANES):
            r = rbase + k
            u0 = urows[r, pl.ds(0, LANES)]
            u1 = urows[r, pl.ds(LANES, LANES)]
            v0 = irows[r, pl.ds(0, LANES)]
            v1 = irows[r, pl.ds(LANES, LANES)]
            s = u0 * v0 + u1 * v1
            acc = jnp.where(lane_iota == k, jnp.sum(s), acc)
        outv[pl.ds(rbase, LANES)] = acc
        return carry

    lax.fori_loop(0, B_PER_W // LANES, body, 0)

    pltpu.sync_copy(outv, out_h.at[pl.ds(base, B_PER_W)])


def kernel(user_table, item_table, user_ids, item_ids):
    uid = user_ids.reshape(NUM_WORKERS * NCHUNKS, CHUNK)
    iid = item_ids.reshape(NUM_WORKERS * NCHUNKS, CHUNK)
    out = _sc_dot(user_table, item_table, uid, iid)
    return out.reshape(BATCH, 1)


# speedup vs baseline: 3.5754x; 3.5754x over previous
import functools

import jax
import jax.numpy as jnp
from jax import lax
from jax.experimental import pallas as pl
from jax.experimental.pallas import tpu as pltpu
from jax.experimental.pallas import tpu_sc as plsc

BATCH = 16384
VOCAB_SIZE = 1000000
EMBED = 32
LANES = 16
NUM_CORES = 2
NUM_SUBCORES = 16
NUM_WORKERS = NUM_CORES * NUM_SUBCORES
B_PER_W = BATCH // NUM_WORKERS
WAVE = 8
NITERS = B_PER_W // LANES


@functools.partial(
    pl.kernel,
    mesh=plsc.VectorSubcoreMesh(core_axis_name="c", subcore_axis_name="s"),
    out_type=jax.ShapeDtypeStruct((BATCH,), jnp.float32),
    compiler_params=pltpu.CompilerParams(needs_layout_passes=False),
    scratch_types=[
        pltpu.VMEM((B_PER_W,), jnp.int32),
        pltpu.VMEM((B_PER_W,), jnp.int32),
        pltpu.VMEM((WAVE * EMBED, 128), jnp.float32),
        pltpu.VMEM((WAVE * EMBED, 128), jnp.float32),
        pltpu.VMEM((B_PER_W,), jnp.float32),
        pltpu.SemaphoreType.DMA,
        pltpu.SemaphoreType.DMA,
    ],
)
def _sc_dot(user_tt, item_tt, uid_h, iid_h, out_h,
            uids, iids, ublk, iblk, outv, sem_id, sem):
    wid = lax.axis_index("s") * NUM_CORES + lax.axis_index("c")
    base = wid * B_PER_W

    pltpu.async_copy(uid_h.at[pl.ds(base, B_PER_W)], uids, sem_id).wait()
    pltpu.async_copy(iid_h.at[pl.ds(base, B_PER_W)], iids, sem_id).wait()

    lane_iota = lax.iota(jnp.int32, LANES)

    def body(it, carry):
        uvec = uids[pl.ds(it * LANES, LANES)]
        ivec = iids[pl.ds(it * LANES, LANES)]
        acc = jnp.zeros((LANES,), jnp.float32)
        for half in range(2):
            copies = []
            for k in range(WAVE):
                kk = half * WAVE + k
                ual = pl.multiple_of((uvec[kk] >> 7) << 7, 128)
                ial = pl.multiple_of((ivec[kk] >> 7) << 7, 128)
                copies.append(pltpu.async_copy(
                    user_tt.at[:, pl.ds(ual, 128)],
                    ublk.at[pl.ds(k * EMBED, EMBED)], sem))
                copies.append(pltpu.async_copy(
                    item_tt.at[:, pl.ds(ial, 128)],
                    iblk.at[pl.ds(k * EMBED, EMBED)], sem))
            for cp in copies:
                cp.wait()
            for k in range(WAVE):
                kk = half * WAVE + k
                uc = jnp.full((LANES,), uvec[kk] & 127, jnp.int32)
                ic = jnp.full((LANES,), ivec[kk] & 127, jnp.int32)
                r0 = lane_iota + (k * EMBED)
                r1 = r0 + LANES
                u0 = plsc.load_gather(ublk, [r0, uc])
                u1 = plsc.load_gather(ublk, [r1, uc])
                v0 = plsc.load_gather(iblk, [r0, ic])
                v1 = plsc.load_gather(iblk, [r1, ic])
                s = jnp.sum(u0 * v0 + u1 * v1)
                acc = jnp.where(lane_iota == kk, s, acc)
        outv[pl.ds(it * LANES, LANES)] = acc
        return carry

    lax.fori_loop(0, NITERS, body, 0)

    pltpu.sync_copy(outv, out_h.at[pl.ds(base, B_PER_W)])


def kernel(user_table, item_table, user_ids, item_ids):
    utt = user_table.T
    itt = item_table.T
    uid = user_ids.reshape(BATCH)
    iid = item_ids.reshape(BATCH)
    out = _sc_dot(utt, itt, uid, iid)
    return out.reshape(BATCH, 1)


# speedup vs baseline: 3.6636x; 1.0247x over previous
import functools

import jax
import jax.numpy as jnp
from jax import lax
from jax.experimental import pallas as pl
from jax.experimental.pallas import tpu as pltpu
from jax.experimental.pallas import tpu_sc as plsc

BATCH = 16384
VOCAB_SIZE = 1000000
EMBED = 32
LANES = 16
NUM_CORES = 2
NUM_SUBCORES = 16
NUM_WORKERS = NUM_CORES * NUM_SUBCORES
B_PER_W = BATCH // NUM_WORKERS
WAVE = 4
NWAVES = B_PER_W // WAVE
SLOTS = 2 * WAVE


@functools.partial(
    pl.kernel,
    mesh=plsc.VectorSubcoreMesh(core_axis_name="c", subcore_axis_name="s"),
    out_type=jax.ShapeDtypeStruct((BATCH,), jnp.float32),
    compiler_params=pltpu.CompilerParams(needs_layout_passes=False),
    scratch_types=[
        pltpu.VMEM((B_PER_W + LANES,), jnp.int32),
        pltpu.VMEM((B_PER_W + LANES,), jnp.int32),
        pltpu.VMEM((SLOTS * EMBED, 128), jnp.float32),
        pltpu.VMEM((SLOTS * EMBED, 128), jnp.float32),
        pltpu.VMEM((B_PER_W,), jnp.float32),
        pltpu.SemaphoreType.DMA,
        pltpu.SemaphoreType.DMA,
        pltpu.SemaphoreType.DMA,
    ],
)
def _sc_dot(user_tt, item_tt, uid_h, iid_h, out_h,
            uids, iids, ublk, iblk, outv, sem_id, sem0, sem1):
    wid = lax.axis_index("s") * NUM_CORES + lax.axis_index("c")
    base = wid * B_PER_W

    pltpu.async_copy(
        uid_h.at[pl.ds(base, B_PER_W)], uids.at[pl.ds(0, B_PER_W)],
        sem_id).wait()
    pltpu.async_copy(
        iid_h.at[pl.ds(base, B_PER_W)], iids.at[pl.ds(0, B_PER_W)],
        sem_id).wait()

    lane_iota = lax.iota(jnp.int32, LANES)

    def fire_wave(w, phase_sem, phase):
        uvec = uids[pl.ds(w * WAVE, LANES)]
        ivec = iids[pl.ds(w * WAVE, LANES)]
        copies = []
        for k in range(WAVE):
            slot = phase * WAVE + k
            ual = pl.multiple_of((uvec[k] >> 7) << 7, 128)
            ial = pl.multiple_of((ivec[k] >> 7) << 7, 128)
            copies.append(pltpu.async_copy(
                user_tt.at[:, pl.ds(ual, 128)],
                ublk.at[pl.ds(slot * EMBED, EMBED)], phase_sem))
            copies.append(pltpu.async_copy(
                item_tt.at[:, pl.ds(ial, 128)],
                iblk.at[pl.ds(slot * EMBED, EMBED)], phase_sem))
        return copies, uvec, ivec

    fire_wave(0, sem0, 0)

    def pair_body(p, acc):
        for phase in range(2):
            w = p * 2 + phase
            psem = (sem0, sem1)[phase]
            nsem = (sem0, sem1)[1 - phase]

            @pl.when(w + 1 < NWAVES)
            def _():
                fire_wave(w + 1, nsem, 1 - phase)

            uvec = uids[pl.ds(w * WAVE, LANES)]
            ivec = iids[pl.ds(w * WAVE, LANES)]
            for k in range(WAVE):
                slot = phase * WAVE + k
                pltpu.make_async_copy(
                    user_tt.at[:, pl.ds(0, 128)],
                    ublk.at[pl.ds(slot * EMBED, EMBED)], psem).wait()
                pltpu.make_async_copy(
                    item_tt.at[:, pl.ds(0, 128)],
                    iblk.at[pl.ds(slot * EMBED, EMBED)], psem).wait()
            for k in range(WAVE):
                slot = phase * WAVE + k
                kk = (w % 4) * WAVE + k
                uc = jnp.full((LANES,), uvec[k] & 127, jnp.int32)
                ic = jnp.full((LANES,), ivec[k] & 127, jnp.int32)
                r0 = lane_iota + (slot * EMBED)
                r1 = r0 + LANES
                u0 = plsc.load_gather(ublk, [r0, uc])
                u1 = plsc.load_gather(ublk, [r1, uc])
                v0 = plsc.load_gather(iblk, [r0, ic])
                v1 = plsc.load_gather(iblk, [r1, ic])
                s = jnp.sum(u0 * v0 + u1 * v1)
                acc = jnp.where(lane_iota == kk, s, acc)

            @pl.when(w % 4 == 3)
            def _():
                outv[pl.ds((w // 4) * LANES, LANES)] = acc
        return acc

    lax.fori_loop(0, NWAVES // 2, pair_body, jnp.zeros((LANES,), jnp.float32))

    pltpu.sync_copy(outv, out_h.at[pl.ds(base, B_PER_W)])


def kernel(user_table, item_table, user_ids, item_ids):
    utt = user_table.T
    itt = item_table.T
    uid = user_ids.reshape(BATCH)
    iid = item_ids.reshape(BATCH)
    out = _sc_dot(utt, itt, uid, iid)
    return out.reshape(BATCH, 1)
